# Initial kernel scaffold; baseline (speedup 1.0000x reference)
import jax
import jax.numpy as jnp
from jax.experimental import pallas as pl


def kernel(x, adj, W1, a_src1, a_trg1, b1, Ws1, W2, a_src2, a_trg2, b2, Ws2):
    raise NotImplementedError("write your pallas kernel here")


# speedup vs baseline: 33.2076x; 33.2076x over previous
import functools

import jax
import jax.numpy as jnp
from jax import lax
from jax.experimental import pallas as pl
from jax.experimental.pallas import tpu as pltpu
from jax.experimental.pallas import tpu_sc as plsc

N = 10000
E = 320000
DH = 128
ROW = DH + 16
NC = 2
NS = 16
LANES = 16
CH = 128
EPT = E // NS
NCHUNK = -(-EPT // CH)
NPAD = 10112
RPT = NPAD // NS
EPAD = E + (NCHUNK * CH - EPT) + CH
EPS = 1e-16
BN = 1000


def _front_body(x_ref, w_ref, asrc_ref, atrg_ref, table_ref, strg_ref):
    xb = x_ref[...]
    p = jnp.dot(xb, w_ref[...], preferred_element_type=jnp.float32)
    sts = []
    zpad = jnp.zeros((xb.shape[0], ROW - DH - 1), jnp.float32)
    for c in range(2):
        pc = p[:, c * DH:(c + 1) * DH]
        ss = jnp.sum(pc * asrc_ref[c, :][None, :], axis=1, keepdims=True)
        st = jnp.sum(pc * atrg_ref[c, :][None, :], axis=1)
        table_ref[c, :, 0:DH] = pc
        table_ref[c, :, DH:DH + 1] = ss
        table_ref[c, :, DH + 1:ROW] = zpad
        sts.append(st)
    z = jnp.zeros_like(sts[0])
    strg_ref[...] = jnp.stack(sts + [z] * 6, axis=1)


def _mid_body(agg_ref, x_ref, b_ref, w_ref, asrc_ref, atrg_ref, ws_ref,
              table_ref, strg_ref, skip_ref):
    xb = x_ref[...]
    hs = []
    for c in range(2):
        num = agg_ref[c, :, 0:DH]
        den = agg_ref[c, :, DH:DH + 1]
        hs.append(num / (den + EPS) + xb + b_ref[c, :][None, :])
    h = jnp.concatenate(hs, axis=1)
    p = jnp.dot(h, w_ref[...], preferred_element_type=jnp.float32)
    sk = jnp.dot(h, ws_ref[...], preferred_element_type=jnp.float32)
    sts = []
    zpad = jnp.zeros((xb.shape[0], ROW - DH - 1), jnp.float32)
    for c in range(2):
        pc = p[:, c * DH:(c + 1) * DH]
        ss = jnp.sum(pc * asrc_ref[c, :][None, :], axis=1, keepdims=True)
        st = jnp.sum(pc * atrg_ref[c, :][None, :], axis=1)
        table_ref[c, :, 0:DH] = pc
        table_ref[c, :, DH:DH + 1] = ss
        table_ref[c, :, DH + 1:ROW] = zpad
        skip_ref[c, :, :] = sk[:, c * DH:(c + 1) * DH]
        sts.append(st)
    z = jnp.zeros_like(sts[0])
    strg_ref[...] = jnp.stack(sts + [z] * 6, axis=1)


def _final_body(agg_ref, skip_ref, b_ref, out_ref):
    for c in range(2):
        num = agg_ref[c, :, 0:DH]
        den = agg_ref[c, :, DH:DH + 1]
        out_ref[:, c * DH:(c + 1) * DH] = (
            num / (den + EPS) + skip_ref[c, :, :] + b_ref[c, :][None, :])


def _make_tc_front(d_in, interpret=False):
    return pl.pallas_call(
        _front_body,
        grid=(N // BN,),
        in_specs=[
            pl.BlockSpec((BN, d_in), lambda i: (i, 0)),
            pl.BlockSpec((d_in, 2 * DH), lambda i: (0, 0)),
            pl.BlockSpec((2, DH), lambda i: (0, 0)),
            pl.BlockSpec((2, DH), lambda i: (0, 0)),
        ],
        out_specs=[
            pl.BlockSpec((2, BN, ROW), lambda i: (0, i, 0)),
            pl.BlockSpec((BN, 8), lambda i: (i, 0)),
        ],
        out_shape=[
            jax.ShapeDtypeStruct((2, N, ROW), jnp.float32),
            jax.ShapeDtypeStruct((N, 8), jnp.float32),
        ],
        interpret=interpret,
    )


def _make_tc_mid(interpret=False):
    return pl.pallas_call(
        _mid_body,
        grid=(N // BN,),
        in_specs=[
            pl.BlockSpec((2, BN, ROW), lambda i: (0, i, 0)),
            pl.BlockSpec((BN, DH), lambda i: (i, 0)),
            pl.BlockSpec((2, DH), lambda i: (0, 0)),
            pl.BlockSpec((2 * DH, 2 * DH), lambda i: (0, 0)),
            pl.BlockSpec((2, DH), lambda i: (0, 0)),
            pl.BlockSpec((2, DH), lambda i: (0, 0)),
            pl.BlockSpec((2 * DH, 2 * DH), lambda i: (0, 0)),
        ],
        out_specs=[
            pl.BlockSpec((2, BN, ROW), lambda i: (0, i, 0)),
            pl.BlockSpec((BN, 8), lambda i: (i, 0)),
            pl.BlockSpec((2, BN, DH), lambda i: (0, i, 0)),
        ],
        out_shape=[
            jax.ShapeDtypeStruct((2, N, ROW), jnp.float32),
            jax.ShapeDtypeStruct((N, 8), jnp.float32),
            jax.ShapeDtypeStruct((2, N, DH), jnp.float32),
        ],
        interpret=interpret,
    )


def _make_tc_final(interpret=False):
    return pl.pallas_call(
        _final_body,
        grid=(N // BN,),
        in_specs=[
            pl.BlockSpec((2, BN, ROW), lambda i: (0, i, 0)),
            pl.BlockSpec((2, BN, DH), lambda i: (0, i, 0)),
            pl.BlockSpec((2, DH), lambda i: (0, 0)),
        ],
        out_specs=pl.BlockSpec((BN, 2 * DH), lambda i: (i, 0)),
        out_shape=jax.ShapeDtypeStruct((N, 2 * DH), jnp.float32),
        interpret=interpret,
    )


def _sc_body(table_ref, strg_ref, src_ref, trg_ref, agg_ref,
             rows, srcb, trgb, ridx, tidx, strgv, exb, agg_sh):
    c = lax.axis_index("c")
    s = lax.axis_index("s")
    coff = c * N
    ebase = s * EPT
    iota = lax.iota(jnp.int32, LANES)
    col_w = jnp.full((LANES,), DH, jnp.int32)
    zero16 = jnp.zeros((LANES,), jnp.float32)

    def _zrow(i, _):
        for j in range(ROW // LANES):
            rows[i, pl.ds(j * LANES, LANES)] = zero16
        return 0
    lax.fori_loop(0, CH, _zrow, 0)
    nfull = RPT // CH
    rem = RPT - nfull * CH
    for k in range(nfull):
        pltpu.sync_copy(rows, agg_sh.at[pl.ds(s * RPT + k * CH, CH)])
    pltpu.sync_copy(rows.at[0:rem], agg_sh.at[pl.ds(s * RPT + nfull * CH, rem)])
    plsc.subcore_barrier()

    def _chunk(k, _):
        base = ebase + k * CH
        pltpu.sync_copy(src_ref.at[pl.ds(c * EPAD + base, CH)], srcb)
        pltpu.sync_copy(trg_ref.at[pl.ds(c * EPAD + base, CH)], trgb.at[0])

        def _bidx(i, _):
            sv = srcb[pl.ds(i * LANES, LANES)]
            tv = trgb[0, pl.ds(i * LANES, LANES)]
            ridx[pl.ds(i * LANES, LANES)] = sv + coff
            tidx[pl.ds(i * LANES, LANES)] = tv + coff
            return 0
        lax.fori_loop(0, CH // LANES, _bidx, 0)

        pltpu.sync_copy(table_ref.at[ridx], rows)
        pltpu.sync_copy(strg_ref.at[tidx], strgv)

        edge0 = k * CH

        def _grp(i, _):
            rid = iota + i * LANES
            ss = plsc.load_gather(rows, [rid, col_w])
            st = strgv[pl.ds(i * LANES, LANES)]
            sc = ss + st
            sc = jnp.where(sc >= 0.0, sc, 0.2 * sc)
            ex = jnp.exp(sc)
            ex = jnp.where(edge0 + rid < EPT, ex, 0.0)
            exb[pl.ds(i * LANES, LANES)] = ex
            plsc.store_scatter(rows, [rid, col_w], ex)
            return 0
        lax.fori_loop(0, CH // LANES, _grp, 0)

        def _scale(e, _):
            exs = plsc.load_gather(exb, [jnp.full((LANES,), 0, jnp.int32) + e])
            for j in range(DH // LANES):
                rows[e, pl.ds(j * LANES, LANES)] = (
                    rows[e, pl.ds(j * LANES, LANES)] * exs)
            return 0
        lax.fori_loop(0, CH, _scale, 0)

        pltpu.sync_copy(rows, agg_sh.at[trgb.at[0]], add=True)
        return 0
    lax.fori_loop(0, NCHUNK, _chunk, 0)

    plsc.subcore_barrier()
    for k in range(nfull):
        pltpu.sync_copy(agg_sh.at[pl.ds(s * RPT + k * CH, CH)],
                        agg_ref.at[c, pl.ds(s * RPT + k * CH, CH)])
    pltpu.sync_copy(agg_sh.at[pl.ds(s * RPT + nfull * CH, rem)],
                    agg_ref.at[c, pl.ds(s * RPT + nfull * CH, rem)])


def _make_sc_edges(interpret=False):
    return pl.kernel(
        _sc_body,
        out_type=jax.ShapeDtypeStruct((2, NPAD, ROW), jnp.float32),
        mesh=plsc.VectorSubcoreMesh(
            core_axis_name="c", subcore_axis_name="s",
            num_cores=NC, num_subcores=NS),
        scratch_types=[
            pltpu.VMEM((CH, ROW), jnp.float32),
            pltpu.VMEM((CH,), jnp.int32),
            pltpu.VMEM((1, CH), jnp.int32),
            pltpu.VMEM((CH,), jnp.int32),
            pltpu.VMEM((CH,), jnp.int32),
            pltpu.VMEM((CH,), jnp.float32),
            pltpu.VMEM((CH,), jnp.float32),
            pltpu.VMEM_SHARED((NPAD, ROW), jnp.float32),
        ],
        compiler_params=pltpu.CompilerParams(use_tc_tiling_on_sc=False, needs_layout_passes=False),
        interpret=interpret,
    )


def kernel(x, adj, W1, a_src1, a_trg1, b1, Ws1, W2, a_src2, a_trg2, b2, Ws2):
    pad = jnp.zeros((2, NCHUNK * CH - EPT + CH), jnp.int32)
    srcp = jnp.concatenate([adj[:, 0, :], pad], axis=1).reshape(-1)
    trgp = jnp.concatenate([adj[:, 1, :], pad], axis=1).reshape(-1)

    tc_front = _make_tc_front(x.shape[1])
    tc_mid = _make_tc_mid()
    tc_final = _make_tc_final()
    sc_edges = _make_sc_edges()

    table1, strg1 = tc_front(x, W1, a_src1[0], a_trg1[0])
    agg1 = sc_edges(table1.reshape(2 * N, ROW), strg1[:, :2].T.reshape(2 * N),
                    srcp, trgp)
    table2, strg2, skip2 = tc_mid(agg1, x, b1.reshape(2, DH), W2,
                                  a_src2[0], a_trg2[0], Ws2)
    agg2 = sc_edges(table2.reshape(2 * N, ROW), strg2[:, :2].T.reshape(2 * N),
                    srcp, trgp)
    return tc_final(agg2, skip2, b2.reshape(2, DH))


# speedup vs baseline: 49.2203x; 1.4822x over previous
import functools

import jax
import jax.numpy as jnp
from jax import lax
from jax.experimental import pallas as pl
from jax.experimental.pallas import tpu as pltpu
from jax.experimental.pallas import tpu_sc as plsc

N = 10000
E = 320000
DH = 128
ROW = DH + 16
NC = 2
NS = 16
LANES = 16
CH = 128
EPT = E // NS
NCHUNK = 162
NPAD = 10112
RPT = NPAD // NS
EPAD = E + (NCHUNK * CH - EPT) + CH
EPS = 1e-16
BN = 1000


def _front_body(x_ref, w_ref, asrc_ref, atrg_ref, table_ref, strg_ref):
    xb = x_ref[...]
    p = jnp.dot(xb, w_ref[...], preferred_element_type=jnp.float32)
    sts = []
    zpad = jnp.zeros((xb.shape[0], ROW - DH - 1), jnp.float32)
    for c in range(2):
        pc = p[:, c * DH:(c + 1) * DH]
        ss = jnp.sum(pc * asrc_ref[c, :][None, :], axis=1, keepdims=True)
        st = jnp.sum(pc * atrg_ref[c, :][None, :], axis=1)
        table_ref[c, :, 0:DH] = pc
        table_ref[c, :, DH:DH + 1] = ss
        table_ref[c, :, DH + 1:ROW] = zpad
        sts.append(st)
    z = jnp.zeros_like(sts[0])
    strg_ref[...] = jnp.stack(sts + [z] * 6, axis=1)


def _mid_body(agg_ref, x_ref, b_ref, w_ref, asrc_ref, atrg_ref, ws_ref,
              table_ref, strg_ref, skip_ref):
    xb = x_ref[...]
    hs = []
    for c in range(2):
        num = agg_ref[c, :, 0:DH]
        den = agg_ref[c, :, DH:DH + 1]
        hs.append(num / (den + EPS) + xb + b_ref[c, :][None, :])
    h = jnp.concatenate(hs, axis=1)
    p = jnp.dot(h, w_ref[...], preferred_element_type=jnp.float32)
    sk = jnp.dot(h, ws_ref[...], preferred_element_type=jnp.float32)
    sts = []
    zpad = jnp.zeros((xb.shape[0], ROW - DH - 1), jnp.float32)
    for c in range(2):
        pc = p[:, c * DH:(c + 1) * DH]
        ss = jnp.sum(pc * asrc_ref[c, :][None, :], axis=1, keepdims=True)
        st = jnp.sum(pc * atrg_ref[c, :][None, :], axis=1)
        table_ref[c, :, 0:DH] = pc
        table_ref[c, :, DH:DH + 1] = ss
        table_ref[c, :, DH + 1:ROW] = zpad
        skip_ref[c, :, :] = sk[:, c * DH:(c + 1) * DH]
        sts.append(st)
    z = jnp.zeros_like(sts[0])
    strg_ref[...] = jnp.stack(sts + [z] * 6, axis=1)


def _final_body(agg_ref, skip_ref, b_ref, out_ref):
    for c in range(2):
        num = agg_ref[c, :, 0:DH]
        den = agg_ref[c, :, DH:DH + 1]
        out_ref[:, c * DH:(c + 1) * DH] = (
            num / (den + EPS) + skip_ref[c, :, :] + b_ref[c, :][None, :])


def _make_tc_front(d_in, interpret=False):
    return pl.pallas_call(
        _front_body,
        grid=(N // BN,),
        in_specs=[
            pl.BlockSpec((BN, d_in), lambda i: (i, 0)),
            pl.BlockSpec((d_in, 2 * DH), lambda i: (0, 0)),
            pl.BlockSpec((2, DH), lambda i: (0, 0)),
            pl.BlockSpec((2, DH), lambda i: (0, 0)),
        ],
        out_specs=[
            pl.BlockSpec((2, BN, ROW), lambda i: (0, i, 0)),
            pl.BlockSpec((BN, 8), lambda i: (i, 0)),
        ],
        out_shape=[
            jax.ShapeDtypeStruct((2, N, ROW), jnp.float32),
            jax.ShapeDtypeStruct((N, 8), jnp.float32),
        ],
        interpret=interpret,
    )


def _make_tc_mid(interpret=False):
    return pl.pallas_call(
        _mid_body,
        grid=(N // BN,),
        in_specs=[
            pl.BlockSpec((2, BN, ROW), lambda i: (0, i, 0)),
            pl.BlockSpec((BN, DH), lambda i: (i, 0)),
            pl.BlockSpec((2, DH), lambda i: (0, 0)),
            pl.BlockSpec((2 * DH, 2 * DH), lambda i: (0, 0)),
            pl.BlockSpec((2, DH), lambda i: (0, 0)),
            pl.BlockSpec((2, DH), lambda i: (0, 0)),
            pl.BlockSpec((2 * DH, 2 * DH), lambda i: (0, 0)),
        ],
        out_specs=[
            pl.BlockSpec((2, BN, ROW), lambda i: (0, i, 0)),
            pl.BlockSpec((BN, 8), lambda i: (i, 0)),
            pl.BlockSpec((2, BN, DH), lambda i: (0, i, 0)),
        ],
        out_shape=[
            jax.ShapeDtypeStruct((2, N, ROW), jnp.float32),
            jax.ShapeDtypeStruct((N, 8), jnp.float32),
            jax.ShapeDtypeStruct((2, N, DH), jnp.float32),
        ],
        interpret=interpret,
    )


def _make_tc_final(interpret=False):
    return pl.pallas_call(
        _final_body,
        grid=(N // BN,),
        in_specs=[
            pl.BlockSpec((2, BN, ROW), lambda i: (0, i, 0)),
            pl.BlockSpec((2, BN, DH), lambda i: (0, i, 0)),
            pl.BlockSpec((2, DH), lambda i: (0, 0)),
        ],
        out_specs=pl.BlockSpec((BN, 2 * DH), lambda i: (i, 0)),
        out_shape=jax.ShapeDtypeStruct((N, 2 * DH), jnp.float32),
        interpret=interpret,
    )


def _sc_body(table_ref, strg_ref, src_ref, trg_ref, agg_ref,
             rows0, rows1, srcb0, srcb1, srcb2, trgb0, trgb1, trgb2,
             ridx0, ridx1, ridx2, tidx0, tidx1, tidx2,
             strgv0, strgv1, exb,
             slin0, slin1, slin2, sgat0, sgat1, ssca0, ssca1,
             agg_sh):
    c = lax.axis_index("c")
    s = lax.axis_index("s")
    coff = c * N
    ebase = s * EPT
    zero16 = jnp.zeros((LANES,), jnp.float32)
    rows = [rows0, rows1]
    srcb = [srcb0, srcb1, srcb2]
    trgb = [trgb0, trgb1, trgb2]
    ridx = [ridx0, ridx1, ridx2]
    tidx = [tidx0, tidx1, tidx2]
    strgv = [strgv0, strgv1]
    slin = [slin0, slin1, slin2]
    sgat = [sgat0, sgat1]
    ssca = [ssca0, ssca1]

    def _zrow(i, _):
        for j in range(ROW // LANES):
            rows0[i, pl.ds(j * LANES, LANES)] = zero16
        return 0
    lax.fori_loop(0, CH, _zrow, 0)
    nfull = RPT // CH
    rem = RPT - nfull * CH
    for k in range(nfull):
        pltpu.sync_copy(rows0, agg_sh.at[pl.ds(s * RPT + k * CH, CH)])
    pltpu.sync_copy(rows0.at[0:rem], agg_sh.at[pl.ds(s * RPT + nfull * CH, rem)])
    plsc.subcore_barrier()

    def fire_lin(k, b):
        base = c * EPAD + ebase + k * CH
        pltpu.async_copy(src_ref.at[pl.ds(base, CH)], srcb[b], slin[b])
        pltpu.async_copy(trg_ref.at[pl.ds(base, CH)], trgb[b].at[0], slin[b])

    def wait_lin(b):
        pltpu.make_async_copy(src_ref.at[pl.ds(0, CH)], srcb[b], slin[b]).wait()
        pltpu.make_async_copy(trg_ref.at[pl.ds(0, CH)], trgb[b].at[0],
                              slin[b]).wait()

    def build_idx(b):
        def _bidx(i, _):
            sv = srcb[b][pl.ds(i * LANES, LANES)]
            tv = trgb[b][0, pl.ds(i * LANES, LANES)]
            ridx[b][pl.ds(i * LANES, LANES)] = sv + coff
            tidx[b][pl.ds(i * LANES, LANES)] = tv + coff
            return 0
        lax.fori_loop(0, CH // LANES, _bidx, 0)

    def fire_gather(bl, br):
        pltpu.async_copy(table_ref.at[ridx[bl]], rows[br], sgat[br])
        pltpu.async_copy(strg_ref.at[tidx[bl]], strgv[br], sgat[br])

    def wait_gather(br):
        pltpu.make_async_copy(table_ref.at[ridx[0]], rows[br], sgat[br]).wait()
        pltpu.make_async_copy(strg_ref.at[tidx[0]], strgv[br], sgat[br]).wait()

    def fire_scatter(bl, br):
        pltpu.async_copy(rows[br], agg_sh.at[trgb[bl].at[0]], ssca[br],
                         add=True)

    def wait_scatter(bl, br):
        pltpu.make_async_copy(rows[br], agg_sh.at[trgb[bl].at[0]],
                              ssca[br]).wait()

    def compute(k, br):
        iota = lax.iota(jnp.int32, LANES)
        col_w = jnp.full((LANES,), DH, jnp.int32)
        edge0 = k * CH

        def _grp(i, _):
            rid = iota + i * LANES
            ss = plsc.load_gather(rows[br], [rid, col_w])
            st = strgv[br][pl.ds(i * LANES, LANES)]
            sc = ss + st
            sc = jnp.where(sc >= 0.0, sc, 0.2 * sc)
            ex = jnp.exp(sc)
            ex = jnp.where(edge0 + rid < EPT, ex, 0.0)
            exb[pl.ds(i * LANES, LANES)] = ex
            plsc.store_scatter(rows[br], [rid, col_w], ex)
            return 0
        lax.fori_loop(0, CH // LANES, _grp, 0)

        def _scale(t, _):
            for u in range(2):
                e = t * 2 + u
                exs = plsc.load_gather(
                    exb, [jnp.full((LANES,), 0, jnp.int32) + e])
                for j in range(DH // LANES):
                    rows[br][e, pl.ds(j * LANES, LANES)] = (
                        rows[br][e, pl.ds(j * LANES, LANES)] * exs)
            return 0
        lax.fori_loop(0, CH // 2, _scale, 0)

    fire_lin(0, 0)
    wait_lin(0)
    build_idx(0)
    fire_gather(0, 0)
    fire_lin(1, 1)

    def _super(i, _):
        for ph in range(6):
            k = i * 6 + ph
            br = ph % 2
            bl = ph % 3
            bl1 = (ph + 1) % 3
            bl2 = (ph + 2) % 3
            br1 = (ph + 1) % 2
            wait_gather(br)
            compute(k, br)
            fire_scatter(bl, br)

            @pl.when(k + 1 < NCHUNK)
            def _():
                wait_lin(bl1)
                build_idx(bl1)

                @pl.when(k >= 1)
                def _():
                    wait_scatter(bl2, br1)
                fire_gather(bl1, br1)

                @pl.when(k + 2 < NCHUNK)
                def _():
                    fire_lin(k + 2, bl2)
        return 0
    lax.fori_loop(0, NCHUNK // 6, _super, 0)
    wait_scatter((NCHUNK - 1) % 3, (NCHUNK - 1) % 2)

    plsc.subcore_barrier()
    for k in range(nfull):
        pltpu.sync_copy(agg_sh.at[pl.ds(s * RPT + k * CH, CH)],
                        agg_ref.at[c, pl.ds(s * RPT + k * CH, CH)])
    pltpu.sync_copy(agg_sh.at[pl.ds(s * RPT + nfull * CH, rem)],
                    agg_ref.at[c, pl.ds(s * RPT + nfull * CH, rem)])


def _make_sc_edges(interpret=False):
    return pl.kernel(
        _sc_body,
        out_type=jax.ShapeDtypeStruct((2, NPAD, ROW), jnp.float32),
        mesh=plsc.VectorSubcoreMesh(
            core_axis_name="c", subcore_axis_name="s",
            num_cores=NC, num_subcores=NS),
        scratch_types=(
            [pltpu.VMEM((CH, ROW), jnp.float32)] * 2 +
            [pltpu.VMEM((CH,), jnp.int32)] * 3 +
            [pltpu.VMEM((1, CH), jnp.int32)] * 3 +
            [pltpu.VMEM((CH,), jnp.int32)] * 3 +
            [pltpu.VMEM((CH,), jnp.int32)] * 3 +
            [pltpu.VMEM((CH,), jnp.float32)] * 2 +
            [pltpu.VMEM((CH,), jnp.float32)] +
            [pltpu.SemaphoreType.DMA] * 7 +
            [pltpu.VMEM_SHARED((NPAD, ROW), jnp.float32)]
        ),
        compiler_params=pltpu.CompilerParams(use_tc_tiling_on_sc=False,
                                             needs_layout_passes=False),
        interpret=interpret,
    )


def kernel(x, adj, W1, a_src1, a_trg1, b1, Ws1, W2, a_src2, a_trg2, b2, Ws2):
    pad = jnp.zeros((2, NCHUNK * CH - EPT + CH), jnp.int32)
    srcp = jnp.concatenate([adj[:, 0, :], pad], axis=1).reshape(-1)
    trgp = jnp.concatenate([adj[:, 1, :], pad], axis=1).reshape(-1)

    tc_front = _make_tc_front(x.shape[1])
    tc_mid = _make_tc_mid()
    tc_final = _make_tc_final()
    sc_edges = _make_sc_edges()

    table1, strg1 = tc_front(x, W1, a_src1[0], a_trg1[0])
    agg1 = sc_edges(table1.reshape(2 * N, ROW), strg1[:, :2].T.reshape(2 * N),
                    srcp, trgp)
    table2, strg2, skip2 = tc_mid(agg1, x, b1.reshape(2, DH), W2,
                                  a_src2[0], a_trg2[0], Ws2)
    agg2 = sc_edges(table2.reshape(2 * N, ROW), strg2[:, :2].T.reshape(2 * N),
                    srcp, trgp)
    return tc_final(agg2, skip2, b2.reshape(2, DH))


# speedup vs baseline: 57.8657x; 1.1756x over previous
import functools

import jax
import jax.numpy as jnp
from jax import lax
from jax.experimental import pallas as pl
from jax.experimental.pallas import tpu as pltpu
from jax.experimental.pallas import tpu_sc as plsc

N = 10000
E = 320000
DH = 128
ROW = DH + 16
NC = 2
NS = 16
LANES = 16
CH = 128
EPT = E // NS
NCHUNK = 162
NPAD = 10112
RPT = NPAD // NS
EPAD = E + (NCHUNK * CH - EPT) + CH
EPS = 1e-16
BN = 1000


def _front_body(x_ref, w_ref, asrc_ref, atrg_ref, table_ref, strg_ref):
    xb = x_ref[...]
    p = jnp.dot(xb, w_ref[...], preferred_element_type=jnp.float32)
    sts = []
    zpad = jnp.zeros((xb.shape[0], ROW - DH - 1), jnp.float32)
    for c in range(2):
        pc = p[:, c * DH:(c + 1) * DH]
        ss = jnp.sum(pc * asrc_ref[c, :][None, :], axis=1, keepdims=True)
        st = jnp.sum(pc * atrg_ref[c, :][None, :], axis=1)
        table_ref[c, :, 0:DH] = pc
        table_ref[c, :, DH:DH + 1] = ss
        table_ref[c, :, DH + 1:ROW] = zpad
        sts.append(st)
    z = jnp.zeros_like(sts[0])
    strg_ref[...] = jnp.stack(sts + [z] * 6, axis=1)


def _mid_body(agg_ref, x_ref, b_ref, w_ref, asrc_ref, atrg_ref, ws_ref,
              table_ref, strg_ref, skip_ref):
    xb = x_ref[...]
    hs = []
    for c in range(2):
        num = agg_ref[c, :, 0:DH]
        den = agg_ref[c, :, DH:DH + 1]
        hs.append(num / (den + EPS) + xb + b_ref[c, :][None, :])
    h = jnp.concatenate(hs, axis=1)
    p = jnp.dot(h, w_ref[...], preferred_element_type=jnp.float32)
    sk = jnp.dot(h, ws_ref[...], preferred_element_type=jnp.float32)
    sts = []
    zpad = jnp.zeros((xb.shape[0], ROW - DH - 1), jnp.float32)
    for c in range(2):
        pc = p[:, c * DH:(c + 1) * DH]
        ss = jnp.sum(pc * asrc_ref[c, :][None, :], axis=1, keepdims=True)
        st = jnp.sum(pc * atrg_ref[c, :][None, :], axis=1)
        table_ref[c, :, 0:DH] = pc
        table_ref[c, :, DH:DH + 1] = ss
        table_ref[c, :, DH + 1:ROW] = zpad
        skip_ref[c, :, :] = sk[:, c * DH:(c + 1) * DH]
        sts.append(st)
    z = jnp.zeros_like(sts[0])
    strg_ref[...] = jnp.stack(sts + [z] * 6, axis=1)


def _final_body(agg_ref, skip_ref, b_ref, out_ref):
    for c in range(2):
        num = agg_ref[c, :, 0:DH]
        den = agg_ref[c, :, DH:DH + 1]
        out_ref[:, c * DH:(c + 1) * DH] = (
            num / (den + EPS) + skip_ref[c, :, :] + b_ref[c, :][None, :])


def _make_tc_front(d_in, interpret=False):
    return pl.pallas_call(
        _front_body,
        grid=(N // BN,),
        in_specs=[
            pl.BlockSpec((BN, d_in), lambda i: (i, 0)),
            pl.BlockSpec((d_in, 2 * DH), lambda i: (0, 0)),
            pl.BlockSpec((2, DH), lambda i: (0, 0)),
            pl.BlockSpec((2, DH), lambda i: (0, 0)),
        ],
        out_specs=[
            pl.BlockSpec((2, BN, ROW), lambda i: (0, i, 0)),
            pl.BlockSpec((BN, 8), lambda i: (i, 0)),
        ],
        out_shape=[
            jax.ShapeDtypeStruct((2, N, ROW), jnp.float32),
            jax.ShapeDtypeStruct((N, 8), jnp.float32),
        ],
        interpret=interpret,
    )


def _make_tc_mid(interpret=False):
    return pl.pallas_call(
        _mid_body,
        grid=(N // BN,),
        in_specs=[
            pl.BlockSpec((2, BN, ROW), lambda i: (0, i, 0)),
            pl.BlockSpec((BN, DH), lambda i: (i, 0)),
            pl.BlockSpec((2, DH), lambda i: (0, 0)),
            pl.BlockSpec((2 * DH, 2 * DH), lambda i: (0, 0)),
            pl.BlockSpec((2, DH), lambda i: (0, 0)),
            pl.BlockSpec((2, DH), lambda i: (0, 0)),
            pl.BlockSpec((2 * DH, 2 * DH), lambda i: (0, 0)),
        ],
        out_specs=[
            pl.BlockSpec((2, BN, ROW), lambda i: (0, i, 0)),
            pl.BlockSpec((BN, 8), lambda i: (i, 0)),
            pl.BlockSpec((2, BN, DH), lambda i: (0, i, 0)),
        ],
        out_shape=[
            jax.ShapeDtypeStruct((2, N, ROW), jnp.float32),
            jax.ShapeDtypeStruct((N, 8), jnp.float32),
            jax.ShapeDtypeStruct((2, N, DH), jnp.float32),
        ],
        interpret=interpret,
    )


def _make_tc_final(interpret=False):
    return pl.pallas_call(
        _final_body,
        grid=(N // BN,),
        in_specs=[
            pl.BlockSpec((2, BN, ROW), lambda i: (0, i, 0)),
            pl.BlockSpec((2, BN, DH), lambda i: (0, i, 0)),
            pl.BlockSpec((2, DH), lambda i: (0, 0)),
        ],
        out_specs=pl.BlockSpec((BN, 2 * DH), lambda i: (i, 0)),
        out_shape=jax.ShapeDtypeStruct((N, 2 * DH), jnp.float32),
        interpret=interpret,
    )


def _sc_body(table_ref, strg_ref, src_ref, trg_ref, agg_ref,
             rows0, rows1, srcb0, srcb1, srcb2, trgb0, trgb1, trgb2,
             ridx0, ridx1, ridx2, tidx0, tidx1, tidx2,
             strgv0, strgv1, exb,
             slin0, slin1, slin2, sgat0, sgat1, ssca0, ssca1,
             agg_sh):
    c = lax.axis_index("c")
    s = lax.axis_index("s")
    coff = c * N
    ebase = s * EPT
    zero16 = jnp.zeros((LANES,), jnp.float32)
    rows = [rows0, rows1]
    srcb = [srcb0, srcb1, srcb2]
    trgb = [trgb0, trgb1, trgb2]
    ridx = [ridx0, ridx1, ridx2]
    tidx = [tidx0, tidx1, tidx2]
    strgv = [strgv0, strgv1]
    slin = [slin0, slin1, slin2]
    sgat = [sgat0, sgat1]
    ssca = [ssca0, ssca1]

    def _zrow(i, _):
        for j in range(ROW // LANES):
            rows0[i, pl.ds(j * LANES, LANES)] = zero16
        return 0
    lax.fori_loop(0, CH, _zrow, 0)
    nfull = RPT // CH
    rem = RPT - nfull * CH
    for k in range(nfull):
        pltpu.sync_copy(rows0, agg_sh.at[pl.ds(s * RPT + k * CH, CH)])
    pltpu.sync_copy(rows0.at[0:rem], agg_sh.at[pl.ds(s * RPT + nfull * CH, rem)])
    plsc.subcore_barrier()

    def fire_lin(k, b):
        base = c * EPAD + ebase + k * CH
        pltpu.async_copy(src_ref.at[pl.ds(base, CH)], srcb[b], slin[b])
        pltpu.async_copy(trg_ref.at[pl.ds(base, CH)], trgb[b].at[0], slin[b])

    def wait_lin(b):
        pltpu.make_async_copy(src_ref.at[pl.ds(0, CH)], srcb[b], slin[b]).wait()
        pltpu.make_async_copy(trg_ref.at[pl.ds(0, CH)], trgb[b].at[0],
                              slin[b]).wait()

    def build_idx(b):
        @plsc.parallel_loop(0, CH // LANES, 1, unroll=2)
        def _bidx(i):
            sv = srcb[b][pl.ds(i * LANES, LANES)]
            tv = trgb[b][0, pl.ds(i * LANES, LANES)]
            ridx[b][pl.ds(i * LANES, LANES)] = sv + coff
            tidx[b][pl.ds(i * LANES, LANES)] = tv + coff

    def fire_gather(bl, br):
        pltpu.async_copy(table_ref.at[ridx[bl]], rows[br], sgat[br])
        pltpu.async_copy(strg_ref.at[tidx[bl]], strgv[br], sgat[br])

    def wait_gather(br):
        pltpu.make_async_copy(table_ref.at[ridx[0]], rows[br], sgat[br]).wait()
        pltpu.make_async_copy(strg_ref.at[tidx[0]], strgv[br], sgat[br]).wait()

    def fire_scatter(bl, br):
        pltpu.async_copy(rows[br], agg_sh.at[trgb[bl].at[0]], ssca[br],
                         add=True)

    def wait_scatter(bl, br):
        pltpu.make_async_copy(rows[br], agg_sh.at[trgb[bl].at[0]],
                              ssca[br]).wait()

    def compute(k, br):
        iota = lax.iota(jnp.int32, LANES)
        col_w = jnp.full((LANES,), DH, jnp.int32)
        edge0 = k * CH

        @plsc.parallel_loop(0, CH // LANES, 1, unroll=2)
        def _grp(i):
            rid = iota + i * LANES
            ss = plsc.load_gather(rows[br], [rid, col_w])
            st = strgv[br][pl.ds(i * LANES, LANES)]
            sc = ss + st
            sc = jnp.where(sc >= 0.0, sc, 0.2 * sc)
            ex = jnp.exp(sc)
            ex = jnp.where(edge0 + rid < EPT, ex, 0.0)
            exb[pl.ds(i * LANES, LANES)] = ex
            plsc.store_scatter(rows[br], [rid, col_w], ex)

        @plsc.parallel_loop(0, CH, 1, unroll=4)
        def _scale(e):
            exs = plsc.load_gather(exb, [jnp.full((LANES,), 0, jnp.int32) + e])
            for j in range(DH // LANES):
                rows[br][e, pl.ds(j * LANES, LANES)] = (
                    rows[br][e, pl.ds(j * LANES, LANES)] * exs)

    fire_lin(0, 0)
    wait_lin(0)
    build_idx(0)
    fire_gather(0, 0)
    fire_lin(1, 1)

    def _super(i, _):
        for ph in range(6):
            k = i * 6 + ph
            br = ph % 2
            bl = ph % 3
            bl1 = (ph + 1) % 3
            bl2 = (ph + 2) % 3
            br1 = (ph + 1) % 2
            wait_gather(br)
            compute(k, br)
            fire_scatter(bl, br)

            @pl.when(k + 1 < NCHUNK)
            def _():
                wait_lin(bl1)
                build_idx(bl1)

                @pl.when(k >= 1)
                def _():
                    wait_scatter(bl2, br1)
                fire_gather(bl1, br1)

                @pl.when(k + 2 < NCHUNK)
                def _():
                    fire_lin(k + 2, bl2)
        return 0
    lax.fori_loop(0, NCHUNK // 6, _super, 0)
    wait_scatter((NCHUNK - 1) % 3, (NCHUNK - 1) % 2)

    plsc.subcore_barrier()
    for k in range(nfull):
        pltpu.sync_copy(agg_sh.at[pl.ds(s * RPT + k * CH, CH)],
                        agg_ref.at[c, pl.ds(s * RPT + k * CH, CH)])
    pltpu.sync_copy(agg_sh.at[pl.ds(s * RPT + nfull * CH, rem)],
                    agg_ref.at[c, pl.ds(s * RPT + nfull * CH, rem)])


def _make_sc_edges(interpret=False):
    return pl.kernel(
        _sc_body,
        out_type=jax.ShapeDtypeStruct((2, NPAD, ROW), jnp.float32),
        mesh=plsc.VectorSubcoreMesh(
            core_axis_name="c", subcore_axis_name="s",
            num_cores=NC, num_subcores=NS),
        scratch_types=(
            [pltpu.VMEM((CH, ROW), jnp.float32)] * 2 +
            [pltpu.VMEM((CH,), jnp.int32)] * 3 +
            [pltpu.VMEM((1, CH), jnp.int32)] * 3 +
            [pltpu.VMEM((CH,), jnp.int32)] * 3 +
            [pltpu.VMEM((CH,), jnp.int32)] * 3 +
            [pltpu.VMEM((CH,), jnp.float32)] * 2 +
            [pltpu.VMEM((CH,), jnp.float32)] +
            [pltpu.SemaphoreType.DMA] * 7 +
            [pltpu.VMEM_SHARED((NPAD, ROW), jnp.float32)]
        ),
        compiler_params=pltpu.CompilerParams(use_tc_tiling_on_sc=False,
                                             needs_layout_passes=False),
        interpret=interpret,
    )


def kernel(x, adj, W1, a_src1, a_trg1, b1, Ws1, W2, a_src2, a_trg2, b2, Ws2):
    pad = jnp.zeros((2, NCHUNK * CH - EPT + CH), jnp.int32)
    srcp = jnp.concatenate([adj[:, 0, :], pad], axis=1).reshape(-1)
    trgp = jnp.concatenate([adj[:, 1, :], pad], axis=1).reshape(-1)

    tc_front = _make_tc_front(x.shape[1])
    tc_mid = _make_tc_mid()
    tc_final = _make_tc_final()
    sc_edges = _make_sc_edges()

    table1, strg1 = tc_front(x, W1, a_src1[0], a_trg1[0])
    agg1 = sc_edges(table1.reshape(2 * N, ROW), strg1[:, :2].T.reshape(2 * N),
                    srcp, trgp)
    table2, strg2, skip2 = tc_mid(agg1, x, b1.reshape(2, DH), W2,
                                  a_src2[0], a_trg2[0], Ws2)
    agg2 = sc_edges(table2.reshape(2 * N, ROW), strg2[:, :2].T.reshape(2 * N),
                    srcp, trgp)
    return tc_final(agg2, skip2, b2.reshape(2, DH))


# speedup vs baseline: 67.1781x; 1.1609x over previous
import functools

import jax
import jax.numpy as jnp
from jax import lax
from jax.experimental import pallas as pl
from jax.experimental.pallas import tpu as pltpu
from jax.experimental.pallas import tpu_sc as plsc

N = 10000
E = 320000
DH = 128
ROW = DH
NC = 2
NS = 16
LANES = 16
CH = 128
EPT = E // NS
NCHUNK = 162
NPAD = 10112
RPT = NPAD // NS
EPAD = E + (NCHUNK * CH - EPT) + CH
EPS = 1e-16
BN = 1000


def _front_body(x_ref, w_ref, asrc_ref, atrg_ref, table_ref, s8_ref):
    xb = x_ref[...]
    p = jnp.dot(xb, w_ref[...], preferred_element_type=jnp.float32)
    cols = []
    for c in range(2):
        pc = p[:, c * DH:(c + 1) * DH]
        cols.append(jnp.sum(pc * asrc_ref[c, :][None, :], axis=1))
        table_ref[c, :, :] = pc
    for c in range(2):
        pc = p[:, c * DH:(c + 1) * DH]
        cols.append(jnp.sum(pc * atrg_ref[c, :][None, :], axis=1))
    z = jnp.zeros_like(cols[0])
    s8_ref[...] = jnp.stack(cols + [z] * 4, axis=1)


def _mid_body(agg_ref, x_ref, b_ref, w_ref, asrc_ref, atrg_ref, ws_ref,
              table_ref, s8_ref, skip_ref):
    xb = x_ref[...]
    hs = []
    for c in range(2):
        hs.append(agg_ref[c, :, :] + xb + b_ref[c, :][None, :])
    h = jnp.concatenate(hs, axis=1)
    p = jnp.dot(h, w_ref[...], preferred_element_type=jnp.float32)
    sk = jnp.dot(h, ws_ref[...], preferred_element_type=jnp.float32)
    cols = []
    for c in range(2):
        pc = p[:, c * DH:(c + 1) * DH]
        cols.append(jnp.sum(pc * asrc_ref[c, :][None, :], axis=1))
        table_ref[c, :, :] = pc
        skip_ref[c, :, :] = sk[:, c * DH:(c + 1) * DH]
    for c in range(2):
        pc = p[:, c * DH:(c + 1) * DH]
        cols.append(jnp.sum(pc * atrg_ref[c, :][None, :], axis=1))
    z = jnp.zeros_like(cols[0])
    s8_ref[...] = jnp.stack(cols + [z] * 4, axis=1)


def _final_body(agg_ref, skip_ref, b_ref, out_ref):
    for c in range(2):
        out_ref[:, c * DH:(c + 1) * DH] = (
            agg_ref[c, :, :] + skip_ref[c, :, :] + b_ref[c, :][None, :])


def _make_tc_front(d_in, interpret=False):
    return pl.pallas_call(
        _front_body,
        grid=(N // BN,),
        in_specs=[
            pl.BlockSpec((BN, d_in), lambda i: (i, 0)),
            pl.BlockSpec((d_in, 2 * DH), lambda i: (0, 0)),
            pl.BlockSpec((2, DH), lambda i: (0, 0)),
            pl.BlockSpec((2, DH), lambda i: (0, 0)),
        ],
        out_specs=[
            pl.BlockSpec((2, BN, ROW), lambda i: (0, i, 0)),
            pl.BlockSpec((BN, 8), lambda i: (i, 0)),
        ],
        out_shape=[
            jax.ShapeDtypeStruct((2, N, ROW), jnp.float32),
            jax.ShapeDtypeStruct((N, 8), jnp.float32),
        ],
        interpret=interpret,
    )


def _make_tc_mid(interpret=False):
    return pl.pallas_call(
        _mid_body,
        grid=(N // BN,),
        in_specs=[
            pl.BlockSpec((2, BN, ROW), lambda i: (0, i, 0)),
            pl.BlockSpec((BN, DH), lambda i: (i, 0)),
            pl.BlockSpec((2, DH), lambda i: (0, 0)),
            pl.BlockSpec((2 * DH, 2 * DH), lambda i: (0, 0)),
            pl.BlockSpec((2, DH), lambda i: (0, 0)),
            pl.BlockSpec((2, DH), lambda i: (0, 0)),
            pl.BlockSpec((2 * DH, 2 * DH), lambda i: (0, 0)),
        ],
        out_specs=[
            pl.BlockSpec((2, BN, ROW), lambda i: (0, i, 0)),
            pl.BlockSpec((BN, 8), lambda i: (i, 0)),
            pl.BlockSpec((2, BN, DH), lambda i: (0, i, 0)),
        ],
        out_shape=[
            jax.ShapeDtypeStruct((2, N, ROW), jnp.float32),
            jax.ShapeDtypeStruct((N, 8), jnp.float32),
            jax.ShapeDtypeStruct((2, N, DH), jnp.float32),
        ],
        interpret=interpret,
    )


def _make_tc_final(interpret=False):
    return pl.pallas_call(
        _final_body,
        grid=(N // BN,),
        in_specs=[
            pl.BlockSpec((2, BN, ROW), lambda i: (0, i, 0)),
            pl.BlockSpec((2, BN, DH), lambda i: (0, i, 0)),
            pl.BlockSpec((2, DH), lambda i: (0, 0)),
        ],
        out_specs=pl.BlockSpec((BN, 2 * DH), lambda i: (i, 0)),
        out_shape=jax.ShapeDtypeStruct((N, 2 * DH), jnp.float32),
        interpret=interpret,
    )


def _sc_body(table_ref, ssrc_ref, strg_ref, src_ref, trg_ref,
             agg_ref,
             rows0, rows1, srcb0, srcb1, srcb2, trgb0, trgb1, trgb2,
             ridx0, ridx1, ridx2, tidx0, tidx1, tidx2,
             strgv0, strgv1, exb0, exb1, ssv, dnb,
             slin0, slin1, slin2, sgat0, sgat1, ssca0, ssca1,
             agg_sh, den_sh):
    c = lax.axis_index("c")
    s = lax.axis_index("s")
    coff = c * N
    ebase = s * EPT
    zero16 = jnp.zeros((LANES,), jnp.float32)
    rows = [rows0, rows1]
    srcb = [srcb0, srcb1, srcb2]
    trgb = [trgb0, trgb1, trgb2]
    ridx = [ridx0, ridx1, ridx2]
    tidx = [tidx0, tidx1, tidx2]
    strgv = [strgv0, strgv1]
    exb = [exb0, exb1]
    slin = [slin0, slin1, slin2]
    sgat = [sgat0, sgat1]
    ssca = [ssca0, ssca1]

    pltpu.sync_copy(ssrc_ref.at[pl.ds(c * N, N)], ssv)

    @plsc.parallel_loop(0, CH, 1, unroll=2)
    def _zrow(i):
        for j in range(DH // LANES):
            rows0[i, pl.ds(j * LANES, LANES)] = zero16

    @plsc.parallel_loop(0, CH // LANES, 1)
    def _zdn(i):
        dnb[pl.ds(i * LANES, LANES)] = zero16

    nfull = RPT // CH
    rem = RPT - nfull * CH
    for k in range(nfull):
        pltpu.sync_copy(rows0, agg_sh.at[pl.ds(s * RPT + k * CH, CH)])
        pltpu.sync_copy(dnb, den_sh.at[pl.ds(s * RPT + k * CH, CH)])
    pltpu.sync_copy(rows0.at[0:rem], agg_sh.at[pl.ds(s * RPT + nfull * CH, rem)])
    pltpu.sync_copy(dnb.at[0:rem], den_sh.at[pl.ds(s * RPT + nfull * CH, rem)])
    plsc.subcore_barrier()

    def fire_lin(k, b):
        base = c * EPAD + ebase + k * CH
        pltpu.async_copy(src_ref.at[pl.ds(base, CH)], srcb[b], slin[b])
        pltpu.async_copy(trg_ref.at[pl.ds(base, CH)], trgb[b].at[0], slin[b])

    def wait_lin(b):
        pltpu.make_async_copy(src_ref.at[pl.ds(0, CH)], srcb[b], slin[b]).wait()
        pltpu.make_async_copy(trg_ref.at[pl.ds(0, CH)], trgb[b].at[0],
                              slin[b]).wait()

    def build_idx(b):
        @plsc.parallel_loop(0, CH // LANES, 1, unroll=2)
        def _bidx(i):
            sv = srcb[b][pl.ds(i * LANES, LANES)]
            tv = trgb[b][0, pl.ds(i * LANES, LANES)]
            ridx[b][pl.ds(i * LANES, LANES)] = sv + coff
            tidx[b][pl.ds(i * LANES, LANES)] = tv + coff

    def fire_gather(bl, br):
        pltpu.async_copy(table_ref.at[ridx[bl]], rows[br], sgat[br])
        pltpu.async_copy(strg_ref.at[tidx[bl]], strgv[br], sgat[br])

    def wait_gather(br):
        pltpu.make_async_copy(table_ref.at[ridx[0]], rows[br], sgat[br]).wait()
        pltpu.make_async_copy(strg_ref.at[tidx[0]], strgv[br], sgat[br]).wait()

    def fire_scatter(bl, br):
        pltpu.async_copy(rows[br], agg_sh.at[trgb[bl].at[0]], ssca[br],
                         add=True)
        pltpu.async_copy(exb[br], den_sh.at[trgb[bl].at[0]], ssca[br],
                         add=True)

    def wait_scatter(bl, br):
        pltpu.make_async_copy(rows[br], agg_sh.at[trgb[bl].at[0]],
                              ssca[br]).wait()
        pltpu.make_async_copy(exb[br], den_sh.at[trgb[bl].at[0]],
                              ssca[br]).wait()

    def compute(k, bl, br):
        edge0 = k * CH
        iota = lax.iota(jnp.int32, LANES)

        @plsc.parallel_loop(0, CH // LANES, 1, unroll=2)
        def _grp(i):
            rid = iota + i * LANES
            sv = srcb[bl][pl.ds(i * LANES, LANES)]
            ss = plsc.load_gather(ssv, [sv])
            st = strgv[br][pl.ds(i * LANES, LANES)]
            sc = ss + st
            sc = jnp.where(sc >= 0.0, sc, 0.2 * sc)
            ex = jnp.exp(sc)
            ex = jnp.where(edge0 + rid < EPT, ex, 0.0)
            exb[br][pl.ds(i * LANES, LANES)] = ex

        @plsc.parallel_loop(0, CH, 1, unroll=4)
        def _scale(e):
            exs = plsc.load_gather(exb[br],
                                   [jnp.full((LANES,), 0, jnp.int32) + e])
            for j in range(DH // LANES):
                rows[br][e, pl.ds(j * LANES, LANES)] = (
                    rows[br][e, pl.ds(j * LANES, LANES)] * exs)

    fire_lin(0, 0)
    wait_lin(0)
    build_idx(0)
    fire_gather(0, 0)
    fire_lin(1, 1)

    def _super(i, _):
        for ph in range(6):
            k = i * 6 + ph
            br = ph % 2
            bl = ph % 3
            bl1 = (ph + 1) % 3
            bl2 = (ph + 2) % 3
            br1 = (ph + 1) % 2
            wait_gather(br)
            compute(k, bl, br)
            fire_scatter(bl, br)

            @pl.when(k + 1 < NCHUNK)
            def _():
                wait_lin(bl1)
                build_idx(bl1)

                @pl.when(k >= 1)
                def _():
                    wait_scatter(bl2, br1)
                fire_gather(bl1, br1)

                @pl.when(k + 2 < NCHUNK)
                def _():
                    fire_lin(k + 2, bl2)
        return 0
    lax.fori_loop(0, NCHUNK // 6, _super, 0)
    wait_scatter((NCHUNK - 1) % 3, (NCHUNK - 1) % 2)

    plsc.subcore_barrier()
    nblk = [CH] * nfull + [rem]
    for k, nb in enumerate(nblk):
        r0 = s * RPT + k * CH
        pltpu.sync_copy(agg_sh.at[pl.ds(r0, nb)], rows0.at[0:nb])
        pltpu.sync_copy(den_sh.at[pl.ds(r0, nb)], dnb.at[0:nb])

        @plsc.parallel_loop(0, nb, 1, unroll=2)
        def _div(r):
            d = plsc.load_gather(dnb, [jnp.full((LANES,), 0, jnp.int32) + r])
            dr = 1.0 / (d + EPS)
            for j in range(DH // LANES):
                rows0[r, pl.ds(j * LANES, LANES)] = (
                    rows0[r, pl.ds(j * LANES, LANES)] * dr)
        pltpu.sync_copy(rows0.at[0:nb], agg_ref.at[c, pl.ds(r0, nb)])


def _make_sc_edges(interpret=False):
    return pl.kernel(
        _sc_body,
        out_type=jax.ShapeDtypeStruct((2, NPAD, DH), jnp.float32),
        mesh=plsc.VectorSubcoreMesh(
            core_axis_name="c", subcore_axis_name="s",
            num_cores=NC, num_subcores=NS),
        scratch_types=(
            [pltpu.VMEM((CH, DH), jnp.float32)] * 2 +
            [pltpu.VMEM((CH,), jnp.int32)] * 3 +
            [pltpu.VMEM((1, CH), jnp.int32)] * 3 +
            [pltpu.VMEM((CH,), jnp.int32)] * 3 +
            [pltpu.VMEM((CH,), jnp.int32)] * 3 +
            [pltpu.VMEM((CH,), jnp.float32)] * 2 +
            [pltpu.VMEM((CH,), jnp.float32)] * 2 +
            [pltpu.VMEM((N,), jnp.float32)] +
            [pltpu.VMEM((CH,), jnp.float32)] +
            [pltpu.SemaphoreType.DMA] * 7 +
            [pltpu.VMEM_SHARED((NPAD, DH), jnp.float32)] +
            [pltpu.VMEM_SHARED((NPAD,), jnp.float32)]
        ),
        compiler_params=pltpu.CompilerParams(use_tc_tiling_on_sc=False,
                                             needs_layout_passes=False),
        interpret=interpret,
    )


def kernel(x, adj, W1, a_src1, a_trg1, b1, Ws1, W2, a_src2, a_trg2, b2, Ws2):
    pad = jnp.zeros((2, NCHUNK * CH - EPT + CH), jnp.int32)
    srcp = jnp.concatenate([adj[:, 0, :], pad], axis=1).reshape(-1)
    trgp = jnp.concatenate([adj[:, 1, :], pad], axis=1).reshape(-1)

    tc_front = _make_tc_front(x.shape[1])
    tc_mid = _make_tc_mid()
    tc_final = _make_tc_final()
    sc_edges = _make_sc_edges()

    table1, s81 = tc_front(x, W1, a_src1[0], a_trg1[0])
    agg1 = sc_edges(table1.reshape(2 * N, DH), s81[:, :2].T.reshape(2 * N),
                    s81[:, 2:4].T.reshape(2 * N), srcp, trgp)
    table2, s82, skip2 = tc_mid(agg1, x, b1.reshape(2, DH), W2,
                                a_src2[0], a_trg2[0], Ws2)
    agg2 = sc_edges(table2.reshape(2 * N, DH), s82[:, :2].T.reshape(2 * N),
                    s82[:, 2:4].T.reshape(2 * N), srcp, trgp)
    return tc_final(agg2, skip2, b2.reshape(2, DH))
